# baseline (speedup 1.0000x reference)
import dataclasses
import functools

import jax
import jax.numpy as jnp
from jax import lax
from jax.experimental import pallas as pl
from jax.experimental.pallas import tpu as pltpu
from jax.experimental.pallas import tpu_sc as plsc

MU_CONST = 3.5
NUM_LANES = 16
NUM_CORES = 2
NUM_SUBCORES = 16
NUM_WORKERS = NUM_CORES * NUM_SUBCORES
BATCH_SIZE = 16384
FACTORS = 64
B_PER_W = BATCH_SIZE // NUM_WORKERS
GATHER_CHUNK = 128
CHUNKS = B_PER_W // GATHER_CHUNK


def _sc_kernel(uidx_hbm, iidx_hbm, eu_hbm, ei_hbm, ub_hbm, ib_hbm, out_hbm,
               uidx_v, iidx_v, eu_v, ei_v, ub_v, ib_v, out_v,
               sem_idx, sem_emb, sem_bias):
    wid = lax.axis_index("s") * NUM_CORES + lax.axis_index("c")
    base = wid * B_PER_W

    cu = pltpu.async_copy(uidx_hbm.at[pl.ds(wid * CHUNKS, CHUNKS)], uidx_v, sem_idx)
    ci = pltpu.async_copy(iidx_hbm.at[pl.ds(wid * CHUNKS, CHUNKS)], iidx_v, sem_idx)
    cu.wait()
    ci.wait()

    copies = []
    for j in range(CHUNKS):
        sl = pl.ds(j * GATHER_CHUNK, GATHER_CHUNK)
        copies.append(pltpu.async_copy(
            eu_hbm.at[uidx_v.at[j]], eu_v.at[sl], sem_emb))
        copies.append(pltpu.async_copy(
            ei_hbm.at[iidx_v.at[j]], ei_v.at[sl], sem_emb))
        copies.append(pltpu.async_copy(
            ub_hbm.at[uidx_v.at[j]], ub_v.at[sl], sem_bias))
        copies.append(pltpu.async_copy(
            ib_hbm.at[iidx_v.at[j]], ib_v.at[sl], sem_bias))
    for c in copies:
        c.wait()

    last_lane = lax.iota(jnp.int32, NUM_LANES) == (NUM_LANES - 1)

    @pl.loop(0, B_PER_W)
    def _(r):
        m = (eu_v[r, pl.ds(0, 16)] * ei_v[r, pl.ds(0, 16)]
             + eu_v[r, pl.ds(16, 16)] * ei_v[r, pl.ds(16, 16)]
             + eu_v[r, pl.ds(32, 16)] * ei_v[r, pl.ds(32, 16)]
             + eu_v[r, pl.ds(48, 16)] * ei_v[r, pl.ds(48, 16)])
        c = plsc.cumsum(m)
        plsc.store_compressed(out_v.at[pl.ds(r, NUM_LANES)], c, mask=last_lane)

    @pl.loop(0, B_PER_W, step=NUM_LANES)
    def _(g):
        sl = pl.ds(g, NUM_LANES)
        out_v[sl] = out_v[sl] + ub_v[sl] + ib_v[sl] + MU_CONST

    pltpu.sync_copy(out_v.at[pl.ds(0, B_PER_W)], out_hbm.at[pl.ds(base, B_PER_W)])


def kernel(user_idx, item_idx, embed_user, embed_item, user_bias, item_bias):
    mesh = plsc.VectorSubcoreMesh(core_axis_name="c", subcore_axis_name="s")
    cp = pltpu.CompilerParams()
    if "needs_layout_passes" in pltpu.CompilerParams.__dataclass_fields__:
        cp = dataclasses.replace(cp, needs_layout_passes=False)
    if "use_tc_tiling_on_sc" in pltpu.CompilerParams.__dataclass_fields__:
        cp = dataclasses.replace(cp, use_tc_tiling_on_sc=False)
    run = pl.kernel(
        _sc_kernel,
        compiler_params=cp,
        out_type=jax.ShapeDtypeStruct((BATCH_SIZE,), jnp.float32),
        mesh=mesh,
        scratch_types=[
            pltpu.VMEM((CHUNKS, GATHER_CHUNK), jnp.int32),
            pltpu.VMEM((CHUNKS, GATHER_CHUNK), jnp.int32),
            pltpu.VMEM((B_PER_W, FACTORS), jnp.float32),
            pltpu.VMEM((B_PER_W, FACTORS), jnp.float32),
            pltpu.VMEM((B_PER_W,), jnp.float32),
            pltpu.VMEM((B_PER_W,), jnp.float32),
            pltpu.VMEM((B_PER_W + NUM_LANES,), jnp.float32),
            pltpu.SemaphoreType.DMA,
            pltpu.SemaphoreType.DMA,
            pltpu.SemaphoreType.DMA,
        ],
    )
    return run(
        user_idx.reshape(BATCH_SIZE // GATHER_CHUNK, GATHER_CHUNK),
        item_idx.reshape(BATCH_SIZE // GATHER_CHUNK, GATHER_CHUNK),
        embed_user,
        embed_item,
        user_bias.reshape(-1),
        item_bias.reshape(-1),
    )


# speedup vs baseline: 1.0015x; 1.0015x over previous
import dataclasses

import jax
import jax.numpy as jnp
from jax import lax
from jax.experimental import pallas as pl
from jax.experimental.pallas import tpu as pltpu
from jax.experimental.pallas import tpu_sc as plsc

MU_CONST = 3.5
NUM_LANES = 16
NUM_CORES = 2
NUM_SUBCORES = 16
NUM_WORKERS = NUM_CORES * NUM_SUBCORES
BATCH_SIZE = 16384
FACTORS = 64
B_PER_W = BATCH_SIZE // NUM_WORKERS
GATHER_CHUNK = 128
CHUNKS = B_PER_W // GATHER_CHUNK


def _sc_kernel(uidx_hbm, iidx_hbm, eu_hbm, ei_hbm, ubw_hbm, ibw_hbm, out_hbm,
               uidx_v, iidx_v, uwin_v, iwin_v, ulane_v, ilane_v,
               eu_v, ei_v, ubw_v, ibw_v, out_v,
               sem_idx, sem_emb, sem_bias):
    wid = lax.axis_index("s") * NUM_CORES + lax.axis_index("c")
    base = wid * B_PER_W

    idx_copies = []
    for j in range(CHUNKS):
        src = pl.ds(base + j * GATHER_CHUNK, GATHER_CHUNK)
        idx_copies.append(pltpu.async_copy(uidx_hbm.at[src], uidx_v.at[j], sem_idx))
        idx_copies.append(pltpu.async_copy(iidx_hbm.at[src], iidx_v.at[j], sem_idx))
    for c in idx_copies:
        c.wait()

    @pl.loop(0, B_PER_W, step=NUM_LANES)
    def _(p):
        j = p // GATHER_CHUNK
        sl = pl.ds(p % GATHER_CHUNK, NUM_LANES)
        fl = pl.ds(p, NUM_LANES)
        u = uidx_v[j, sl]
        i = iidx_v[j, sl]
        uwin_v[j, sl] = u >> 4
        iwin_v[j, sl] = i >> 4
        ulane_v[fl] = u & 15
        ilane_v[fl] = i & 15

    copies = []
    for j in range(CHUNKS):
        sl = pl.ds(j * GATHER_CHUNK, GATHER_CHUNK)
        copies.append(pltpu.async_copy(
            eu_hbm.at[uidx_v.at[j]], eu_v.at[sl], sem_emb))
        copies.append(pltpu.async_copy(
            ei_hbm.at[iidx_v.at[j]], ei_v.at[sl], sem_emb))
        copies.append(pltpu.async_copy(
            ubw_hbm.at[uwin_v.at[j]], ubw_v.at[sl], sem_bias))
        copies.append(pltpu.async_copy(
            ibw_hbm.at[iwin_v.at[j]], ibw_v.at[sl], sem_bias))
    for c in copies:
        c.wait()

    lane = lax.iota(jnp.int32, NUM_LANES)
    last_lane = lane == (NUM_LANES - 1)

    @pl.loop(0, B_PER_W)
    def _(r):
        m = (eu_v[r, pl.ds(0, 16)] * ei_v[r, pl.ds(0, 16)]
             + eu_v[r, pl.ds(16, 16)] * ei_v[r, pl.ds(16, 16)]
             + eu_v[r, pl.ds(32, 16)] * ei_v[r, pl.ds(32, 16)]
             + eu_v[r, pl.ds(48, 16)] * ei_v[r, pl.ds(48, 16)])
        c = plsc.cumsum(m)
        plsc.store_compressed(out_v.at[pl.ds(r, NUM_LANES)], c, mask=last_lane)

    @pl.loop(0, B_PER_W, step=NUM_LANES)
    def _(g):
        rows = lane + g
        sl = pl.ds(g, NUM_LANES)
        ub = plsc.load_gather(ubw_v, [rows, ulane_v[sl]])
        ib = plsc.load_gather(ibw_v, [rows, ilane_v[sl]])
        out_v[sl] = out_v[sl] + ub + ib + MU_CONST

    pltpu.sync_copy(out_v.at[pl.ds(0, B_PER_W)], out_hbm.at[pl.ds(base, B_PER_W)])


def kernel(user_idx, item_idx, embed_user, embed_item, user_bias, item_bias):
    mesh = plsc.VectorSubcoreMesh(core_axis_name="c", subcore_axis_name="s")
    cp = pltpu.CompilerParams()
    if "needs_layout_passes" in pltpu.CompilerParams.__dataclass_fields__:
        cp = dataclasses.replace(cp, needs_layout_passes=False)
    if "use_tc_tiling_on_sc" in pltpu.CompilerParams.__dataclass_fields__:
        cp = dataclasses.replace(cp, use_tc_tiling_on_sc=False)
    run = pl.kernel(
        _sc_kernel,
        compiler_params=cp,
        out_type=jax.ShapeDtypeStruct((BATCH_SIZE,), jnp.float32),
        mesh=mesh,
        scratch_types=[
            pltpu.VMEM((CHUNKS, GATHER_CHUNK), jnp.int32),
            pltpu.VMEM((CHUNKS, GATHER_CHUNK), jnp.int32),
            pltpu.VMEM((CHUNKS, GATHER_CHUNK), jnp.int32),
            pltpu.VMEM((CHUNKS, GATHER_CHUNK), jnp.int32),
            pltpu.VMEM((B_PER_W,), jnp.int32),
            pltpu.VMEM((B_PER_W,), jnp.int32),
            pltpu.VMEM((B_PER_W, FACTORS), jnp.float32),
            pltpu.VMEM((B_PER_W, FACTORS), jnp.float32),
            pltpu.VMEM((B_PER_W, NUM_LANES), jnp.float32),
            pltpu.VMEM((B_PER_W, NUM_LANES), jnp.float32),
            pltpu.VMEM((B_PER_W + NUM_LANES,), jnp.float32),
            pltpu.SemaphoreType.DMA,
            pltpu.SemaphoreType.DMA,
            pltpu.SemaphoreType.DMA,
        ],
    )
    return run(user_idx, item_idx, embed_user, embed_item,
               user_bias.reshape(-1, NUM_LANES), item_bias.reshape(-1, NUM_LANES))
